# Initial kernel scaffold; baseline (speedup 1.0000x reference)
import jax
import jax.numpy as jnp
from jax.experimental import pallas as pl


def kernel(x, edge_index, batch, W_emb, b_emb, W1a, b1a, W2a, b2a, epsa, W1b, b1b, W2b, b2b, epsb, W1c, b1c, W2c, b2c, epsc, W_c1, b_c1, W_c2, b_c2):
    raise NotImplementedError("write your pallas kernel here")


# speedup vs baseline: 3.5659x; 3.5659x over previous
import functools
import jax
import jax.numpy as jnp
from jax import lax
from jax.experimental import pallas as pl
from jax.experimental.pallas import tpu as pltpu
from jax.experimental.pallas import tpu_sc as plsc

N = 50000
E = 800000
F_IN = 128
H = 64
HH = 32
C = 10
G = 512

NS = 16
CH = 128
ECHUNKS = E // CH
EITER = -(-ECHUNKS // NS)
NCHUNKS = N // CH
NREM = N - NCHUNKS * CH
NITER = -(-NCHUNKS // NS)
RPTA = 3128
RPTL = N - (NS - 1) * RPTA
ZR = 136
ZFULL = 22
MSEG = G * HH // NS
CSEG = G // NS

_mesh = plsc.VectorSubcoreMesh(core_axis_name="c", subcore_axis_name="s")
_sc_params = pltpu.CompilerParams(use_tc_tiling_on_sc=False,
                                  needs_layout_passes=False)


def _lanes():
    return lax.iota(jnp.int32, 16)


@functools.partial(
    pl.kernel,
    mesh=_mesh,
    compiler_params=_sc_params,
    out_type=jax.ShapeDtypeStruct((2 * N, HH), jnp.float32),
    scratch_types=[
        pltpu.VMEM_SHARED((N, HH), jnp.float32),
        pltpu.VMEM((ZR, HH), jnp.float32),
        pltpu.VMEM((CH,), jnp.int32),
        pltpu.VMEM((CH,), jnp.int32),
        pltpu.VMEM((CH,), jnp.int32),
        pltpu.VMEM((CH, HH), jnp.float32),
    ],
)
def _sc_aggregate(h2, src, dst, out, acc, zbuf, srcb, dstb, adjb, rowsb):
    c = lax.axis_index("c")
    s = lax.axis_index("s")
    lanes = _lanes()
    zero16 = jnp.zeros((16,), jnp.float32)

    for r in range(ZR):
        zbuf[r, pl.ds(0, 16)] = zero16
        zbuf[r, pl.ds(16, 16)] = zero16

    def zc(m, carry):
        pltpu.sync_copy(zbuf, acc.at[pl.ds(s * RPTA + m * ZR, ZR)])
        return carry
    lax.fori_loop(0, ZFULL, zc, 0)

    @pl.when(s < NS - 1)
    def _():
        pltpu.sync_copy(zbuf, acc.at[pl.ds(s * RPTA + ZFULL * ZR, ZR)])

    @pl.when(s == NS - 1)
    def _():
        rem = RPTL - ZFULL * ZR
        pltpu.sync_copy(zbuf.at[pl.ds(0, rem)],
                        acc.at[pl.ds(s * RPTA + ZFULL * ZR, rem)])
    plsc.subcore_barrier()

    cN = c * N

    def body(i, carry):
        k = s + i * NS

        @pl.when(k < ECHUNKS)
        def _():
            base = k * CH
            pltpu.sync_copy(src.at[pl.ds(base, CH)], srcb)
            pltpu.sync_copy(dst.at[pl.ds(base, CH)], dstb)

            def adj(j, c2):
                sl = pl.ds(j * 16, 16)
                adjb[sl] = srcb[sl] + cN
                return c2
            lax.fori_loop(0, CH // 16, adj, 0)
            pltpu.sync_copy(h2.at[adjb], rowsb)
            pltpu.sync_copy(rowsb, acc.at[dstb], add=True)
        return carry
    lax.fori_loop(0, EITER, body, 0)
    plsc.subcore_barrier()

    @pl.when(s < NS - 1)
    def _():
        pltpu.sync_copy(acc.at[pl.ds(s * RPTA, RPTA)],
                        out.at[pl.ds(cN + s * RPTA, RPTA)])

    @pl.when(s == NS - 1)
    def _():
        pltpu.sync_copy(acc.at[pl.ds((NS - 1) * RPTA, RPTL)],
                        out.at[pl.ds(cN + (NS - 1) * RPTA, RPTL)])


@functools.partial(
    pl.kernel,
    mesh=_mesh,
    compiler_params=_sc_params,
    out_type=[
        jax.ShapeDtypeStruct((2 * G, HH), jnp.float32),
        jax.ShapeDtypeStruct((2 * G * HH,), jnp.float32),
        jax.ShapeDtypeStruct((2 * G,), jnp.float32),
    ],
    scratch_types=[
        pltpu.VMEM_SHARED((G, HH), jnp.float32),
        pltpu.VMEM_SHARED((NS * G * HH,), jnp.float32),
        pltpu.VMEM_SHARED((NS * G,), jnp.float32),
        pltpu.VMEM((G * HH,), jnp.float32),
        pltpu.VMEM((G,), jnp.float32),
        pltpu.VMEM((CSEG, HH), jnp.float32),
        pltpu.VMEM((CH, HH), jnp.float32),
        pltpu.VMEM((CH,), jnp.int32),
        pltpu.VMEM((NREM, HH), jnp.float32),
        pltpu.VMEM((NREM,), jnp.int32),
        pltpu.VMEM((MSEG,), jnp.float32),
        pltpu.VMEM((MSEG,), jnp.float32),
        pltpu.VMEM((CSEG,), jnp.float32),
        pltpu.VMEM((CSEG,), jnp.float32),
    ],
)
def _sc_pool(h2, batch, sums, maxs, cnts, ssum, smax, scnt, maxacc, cntacc,
             zbuf2, rowsb, batchb, rrows, rbatch, macc, mtmp, cacc, ctmp):
    c = lax.axis_index("c")
    s = lax.axis_index("s")
    lanes = _lanes()
    neg16 = jnp.full((16,), -jnp.inf, jnp.float32)
    zero16 = jnp.zeros((16,), jnp.float32)
    ones16 = jnp.full((16,), 1.0, jnp.float32)
    mask0 = lanes == 0

    def mi(i, carry):
        maxacc[pl.ds(i * 16, 16)] = neg16
        return carry
    lax.fori_loop(0, G * HH // 16, mi, 0)

    def ci(i, carry):
        cntacc[pl.ds(i * 16, 16)] = zero16
        return carry
    lax.fori_loop(0, G // 16, ci, 0)

    for r in range(CSEG):
        zbuf2[r, pl.ds(0, 16)] = zero16
        zbuf2[r, pl.ds(16, 16)] = zero16
    pltpu.sync_copy(zbuf2, ssum.at[pl.ds(s * CSEG, CSEG)])
    plsc.subcore_barrier()

    cN = c * N

    def process(rows_ref, batch_ref, nrows):
        def row(r, carry):
            rfull = jnp.full((16,), r, jnp.int32)
            b = plsc.load_gather(batch_ref, [rfull])
            for cg in range(HH // 16):
                cols = cg * 16 + lanes
                v = rows_ref[r, pl.ds(cg * 16, 16)]
                idx = b * HH + cols
                old = plsc.load_gather(maxacc, [idx])
                plsc.store_scatter(maxacc, [idx], jnp.maximum(old, v))
            plsc.addupdate_scatter(cntacc, [b], ones16, mask=mask0)
            return carry
        lax.fori_loop(0, nrows, row, 0)
        pltpu.sync_copy(rows_ref, ssum.at[batch_ref], add=True)

    def body(i, carry):
        k = s + i * NS

        @pl.when(k < NCHUNKS)
        def _():
            base = k * CH
            pltpu.sync_copy(h2.at[pl.ds(cN + base, CH)], rowsb)
            pltpu.sync_copy(batch.at[pl.ds(base, CH)], batchb)
            process(rowsb, batchb, CH)
        return carry
    lax.fori_loop(0, NITER, body, 0)

    @pl.when(s == NS - 1)
    def _():
        base = NCHUNKS * CH
        pltpu.sync_copy(h2.at[pl.ds(cN + base, NREM)], rrows)
        pltpu.sync_copy(batch.at[pl.ds(base, NREM)], rbatch)
        process(rrows, rbatch, NREM)

    pltpu.sync_copy(maxacc, smax.at[pl.ds(s * G * HH, G * HH)])
    pltpu.sync_copy(cntacc, scnt.at[pl.ds(s * G, G)])
    plsc.subcore_barrier()

    mbase = s * MSEG
    pltpu.sync_copy(smax.at[pl.ds(mbase, MSEG)], macc)

    def mp(p, carry):
        pltpu.sync_copy(smax.at[pl.ds(p * G * HH + mbase, MSEG)], mtmp)

        def mj(j, c2):
            sl = pl.ds(j * 16, 16)
            macc[sl] = jnp.maximum(macc[sl], mtmp[sl])
            return c2
        lax.fori_loop(0, MSEG // 16, mj, 0)
        return carry
    lax.fori_loop(1, NS, mp, 0)
    pltpu.sync_copy(macc, maxs.at[pl.ds(c * G * HH + mbase, MSEG)])

    cbase = s * CSEG
    pltpu.sync_copy(scnt.at[pl.ds(cbase, CSEG)], cacc)

    def cp(p, carry):
        pltpu.sync_copy(scnt.at[pl.ds(p * G + cbase, CSEG)], ctmp)

        def cj(j, c2):
            sl = pl.ds(j * 16, 16)
            cacc[sl] = cacc[sl] + ctmp[sl]
            return c2
        lax.fori_loop(0, CSEG // 16, cj, 0)
        return carry
    lax.fori_loop(1, NS, cp, 0)
    pltpu.sync_copy(cacc, cnts.at[pl.ds(c * G + cbase, CSEG)])

    pltpu.sync_copy(ssum.at[pl.ds(cbase, CSEG)],
                    sums.at[pl.ds(c * G + cbase, CSEG)])


BLK = 2000
NB = N // BLK


def _emb_body(x_ref, w_ref, b_ref, out_ref):
    z = jnp.dot(x_ref[...], w_ref[...],
                preferred_element_type=jnp.float32) + b_ref[...]
    out_ref[0] = z[:, :HH]
    out_ref[1] = z[:, HH:]


def _tc_emb(x, W, b):
    return pl.pallas_call(
        _emb_body,
        grid=(NB,),
        in_specs=[
            pl.BlockSpec((BLK, F_IN), lambda i: (i, 0)),
            pl.BlockSpec((F_IN, H), lambda i: (0, 0)),
            pl.BlockSpec((1, H), lambda i: (0, 0)),
        ],
        out_specs=pl.BlockSpec((2, BLK, HH), lambda i: (0, i, 0)),
        out_shape=jax.ShapeDtypeStruct((2, N, HH), jnp.float32),
    )(x, W, b.reshape(1, H))


def _gin_body(h_ref, a_ref, w1_ref, b1_ref, w2_ref, b2_ref, eps_ref, out_ref):
    h = jnp.concatenate([h_ref[0], h_ref[1]], axis=1)
    agg = jnp.concatenate([a_ref[0], a_ref[1]], axis=1)
    z = (1.0 + eps_ref[0, 0]) * h + agg
    z = jnp.maximum(jnp.dot(z, w1_ref[...],
                            preferred_element_type=jnp.float32) + b1_ref[...],
                    0.0)
    z = jnp.dot(z, w2_ref[...],
                preferred_element_type=jnp.float32) + b2_ref[...]
    z = jnp.where(z > 0.0, z, jnp.exp(z) - 1.0)
    out_ref[0] = z[:, :HH]
    out_ref[1] = z[:, HH:]


def _tc_gin(h2, agg2, W1, b1, W2, b2, eps):
    return pl.pallas_call(
        _gin_body,
        grid=(NB,),
        in_specs=[
            pl.BlockSpec((2, BLK, HH), lambda i: (0, i, 0)),
            pl.BlockSpec((2, BLK, HH), lambda i: (0, i, 0)),
            pl.BlockSpec((H, 2 * H), lambda i: (0, 0)),
            pl.BlockSpec((1, 2 * H), lambda i: (0, 0)),
            pl.BlockSpec((2 * H, H), lambda i: (0, 0)),
            pl.BlockSpec((1, H), lambda i: (0, 0)),
            pl.BlockSpec((1, 1), lambda i: (0, 0)),
        ],
        out_specs=pl.BlockSpec((2, BLK, HH), lambda i: (0, i, 0)),
        out_shape=jax.ShapeDtypeStruct((2, N, HH), jnp.float32),
    )(h2, agg2, W1, b1.reshape(1, 2 * H), W2, b2.reshape(1, H),
      eps.reshape(1, 1))


def _cls_body(sums_ref, maxs_ref, cnt_ref, w1_ref, b1_ref, w2_ref, b2_ref,
              out_ref):
    inv = 1.0 / jnp.maximum(cnt_ref[...], 1.0)
    mean = jnp.concatenate([sums_ref[:G], sums_ref[G:]], axis=1) * inv
    mx = jnp.concatenate([maxs_ref[:G], maxs_ref[G:]], axis=1)
    feat = jnp.concatenate([mean, mx], axis=1)
    z = jnp.maximum(jnp.dot(feat, w1_ref[...],
                            preferred_element_type=jnp.float32) + b1_ref[...],
                    0.0)
    out_ref[...] = jnp.dot(z, w2_ref[...],
                           preferred_element_type=jnp.float32) + b2_ref[...]


def _tc_cls(sums, maxs, cnt, W1, b1, W2, b2):
    return pl.pallas_call(
        _cls_body,
        out_shape=jax.ShapeDtypeStruct((G, C), jnp.float32),
    )(sums, maxs, cnt, W1, b1.reshape(1, H), W2, b2.reshape(1, C))


def kernel(x, edge_index, batch, W_emb, b_emb, W1a, b1a, W2a, b2a, epsa,
           W1b, b1b, W2b, b2b, epsb, W1c, b1c, W2c, b2c, epsc,
           W_c1, b_c1, W_c2, b_c2):
    src = edge_index[0]
    dst = edge_index[1]

    h2 = _tc_emb(x, W_emb, b_emb)
    for (W1, b1, W2, b2, eps) in (
            (W1a, b1a, W2a, b2a, epsa),
            (W1b, b1b, W2b, b2b, epsb),
            (W1c, b1c, W2c, b2c, epsc)):
        agg = _sc_aggregate(h2.reshape(2 * N, HH), src, dst)
        h2 = _tc_gin(h2, agg.reshape(2, N, HH), W1, b1, W2, b2, eps)

    sums, maxs, cnts = _sc_pool(h2.reshape(2 * N, HH), batch)
    return _tc_cls(sums, maxs.reshape(2 * G, HH),
                   cnts[:G].reshape(G, 1), W_c1, b_c1, W_c2, b_c2)
